# baseline (speedup 1.0000x reference)
import jax
import jax.numpy as jnp
from jax.experimental import pallas as pl
from jax.experimental.pallas import tpu as pltpu

B = 8192
D = 768
L = 256
K = 8192

BB = 2048
KC = 1024
LANES = 128


def _loss_body(x_ref, a_ref, h_ref, cc_ref, out_ref, c2_ref, ccb_ref):
    i = pl.program_id(0)

    @pl.when(i == 0)
    def _():
        cf = cc_ref[...]
        c2 = jnp.sum(cf * cf, axis=1)
        c2_ref[...] = (0.5 * c2).reshape(1, K).astype(jnp.bfloat16)
        ccb_ref[...] = cf.astype(jnp.bfloat16)

    diff = x_ref[...] - a_ref[...]
    recon = jnp.sum(diff * diff)

    h = h_ref[...]
    h2 = jnp.sum(h * h, axis=1)
    hb = h.astype(jnp.bfloat16)

    m = jnp.full((BB, LANES), -jnp.inf, dtype=jnp.bfloat16)
    for kc in range(K // KC):
        s = jax.lax.dot_general(
            hb, ccb_ref[kc * KC:(kc + 1) * KC, :],
            (((1,), (1,)), ((), ())),
            preferred_element_type=jnp.float32,
        )
        sb = s.astype(jnp.bfloat16)
        for t in range(KC // LANES):
            j = kc * KC + t * LANES
            m = jnp.maximum(m, sb[:, t * LANES:(t + 1) * LANES]
                            - c2_ref[0:1, j:j + LANES])
    m_row = jnp.max(m.astype(jnp.float32), axis=1)

    d2 = h2 - 2.0 * m_row
    part = jnp.reshape(recon / (B * D) + jnp.sum(d2) / (B * L), (1, 1))

    @pl.when(i == 0)
    def _():
        out_ref[...] = jnp.zeros((1, 1), jnp.float32)
    out_ref[...] += part


def kernel(x, h_x, a_x, cluster_centers):
    out = pl.pallas_call(
        _loss_body,
        grid=(B // BB,),
        in_specs=[
            pl.BlockSpec((BB, D), lambda i: (i, 0)),
            pl.BlockSpec((BB, D), lambda i: (i, 0)),
            pl.BlockSpec((BB, L), lambda i: (i, 0)),
            pl.BlockSpec((K, L), lambda i: (0, 0)),
        ],
        out_specs=pl.BlockSpec((1, 1), lambda i: (0, 0)),
        out_shape=jax.ShapeDtypeStruct((1, 1), jnp.float32),
        scratch_shapes=[pltpu.VMEM((1, K), jnp.bfloat16),
                        pltpu.VMEM((K, L), jnp.bfloat16)],
    )(x, a_x, h_x, cluster_centers)
    return out[0, 0]


# speedup vs baseline: 1.0574x; 1.0574x over previous
import jax
import jax.numpy as jnp
from jax.experimental import pallas as pl
from jax.experimental.pallas import tpu as pltpu

B = 8192
D = 768
L = 256
K = 8192

BB = 1024
KC = 4096
LANES = 128


def _loss_body(x_ref, a_ref, h_ref, cc_ref, out_ref, c2_ref, ccb_ref):
    i = pl.program_id(0)

    @pl.when(i == 0)
    def _():
        cf = cc_ref[...]
        c2 = jnp.sum(cf * cf, axis=1)
        c2_ref[...] = (0.5 * c2).reshape(1, K).astype(jnp.bfloat16)
        ccb_ref[...] = cf.astype(jnp.bfloat16)

    diff = x_ref[...] - a_ref[...]
    recon = jnp.sum(diff * diff)

    h = h_ref[...]
    h2 = jnp.sum(h * h, axis=1)
    hb = h.astype(jnp.bfloat16)

    m = jnp.full((BB, LANES), -jnp.inf, dtype=jnp.bfloat16)
    for kc in range(K // KC):
        s = jax.lax.dot_general(
            hb, ccb_ref[kc * KC:(kc + 1) * KC, :],
            (((1,), (1,)), ((), ())),
            preferred_element_type=jnp.float32,
        )
        sb = s.astype(jnp.bfloat16)
        for t in range(KC // LANES):
            j = kc * KC + t * LANES
            m = jnp.maximum(m, sb[:, t * LANES:(t + 1) * LANES]
                            - c2_ref[0:1, j:j + LANES])
    m_row = jnp.max(m.astype(jnp.float32), axis=1)

    d2 = h2 - 2.0 * m_row
    part = jnp.reshape(recon / (B * D) + jnp.sum(d2) / (B * L), (1, 1))

    @pl.when(i == 0)
    def _():
        out_ref[...] = jnp.zeros((1, 1), jnp.float32)
    out_ref[...] += part


def kernel(x, h_x, a_x, cluster_centers):
    out = pl.pallas_call(
        _loss_body,
        grid=(B // BB,),
        in_specs=[
            pl.BlockSpec((BB, D), lambda i: (i, 0)),
            pl.BlockSpec((BB, D), lambda i: (i, 0)),
            pl.BlockSpec((BB, L), lambda i: (i, 0)),
            pl.BlockSpec((K, L), lambda i: (0, 0)),
        ],
        out_specs=pl.BlockSpec((1, 1), lambda i: (0, 0)),
        out_shape=jax.ShapeDtypeStruct((1, 1), jnp.float32),
        scratch_shapes=[pltpu.VMEM((1, K), jnp.bfloat16),
                        pltpu.VMEM((K, L), jnp.bfloat16)],
    )(x, a_x, h_x, cluster_centers)
    return out[0, 0]
